# baseline (speedup 1.0000x reference)
import functools

import jax
import jax.numpy as jnp
import numpy as np
from jax import lax
from jax.experimental import pallas as pl
from jax.experimental.pallas import tpu as pltpu
from jax.experimental.pallas import tpu_sc as plsc

NUM_LEVELS = 3
BASE_SCALE = 0.25
RESOLUTION = 3
TRUNCATE_K = 128
KNN = 32

N = 8192
DIM = 128
K = TRUNCATE_K
BLK = 256
MIN32 = np.int32(-2147483648)

NW = 32
RPW = N // NW
NCH = N // 16


def _kth_largest(x, kth):
    bits = lax.bitcast_convert_type(x, jnp.int32)
    key = jnp.where(bits >= 0, bits, ~bits ^ MIN32)
    ones = jnp.ones((x.shape[1], 1), jnp.float32)
    t_u = jnp.zeros((x.shape[0], 1), jnp.int32)
    for bit in range(31, -1, -1):
        m = np.uint32(1 << bit).astype(np.int32)
        cand = t_u | m
        ind = (key >= (cand ^ MIN32)).astype(jnp.float32)
        cnt = lax.dot_general(ind, ones, (((1,), (0,)), ((), ())),
                              preferred_element_type=jnp.float32)
        t_u = jnp.where(cnt >= kth, cand, t_u)
    fbits = jnp.where(t_u < 0, t_u ^ MIN32, ~t_u)
    return lax.bitcast_convert_type(fbits, jnp.float32)


def _s1_body(a_ref, b_ref, corr_ref, th_ref):
    a = a_ref[...]
    b = b_ref[...]
    corr = lax.dot_general(a, b, (((0,), (0,)), ((), ())),
                           preferred_element_type=jnp.float32)
    corr = corr * np.float32(1.0 / np.sqrt(DIM))
    corr_ref[...] = corr
    t = _kth_largest(corr, K)
    th_ref[...] = t.reshape(1, 1, BLK)


def _s1(f1, f2):
    grid = N // BLK
    corr, th = pl.pallas_call(
        _s1_body,
        grid=(grid,),
        in_specs=[
            pl.BlockSpec((DIM, BLK), lambda i: (0, i)),
            pl.BlockSpec((DIM, N), lambda i: (0, 0)),
        ],
        out_specs=[
            pl.BlockSpec((BLK, N), lambda i: (i, 0)),
            pl.BlockSpec((1, 1, BLK), lambda i: (i, 0, 0)),
        ],
        out_shape=[
            jax.ShapeDtypeStruct((N, N), jnp.float32),
            jax.ShapeDtypeStruct((grid, 1, BLK), jnp.float32),
        ],
    )(f1, f2)
    return corr, th.reshape(N)


def _s2_body(corr_hbm, th_hbm, xs_hbm, ys_hbm, zs_hbm,
             vals_hbm, gx_hbm, gy_hbm, gz_hbm,
             row_v, th_v, xs_v, ys_v, zs_v,
             gtv_v, gti_v, eqv_v, eqi_v, gxb_v, gyb_v, gzb_v):
    wid = lax.axis_index("s") * 2 + lax.axis_index("c")
    base = wid * RPW
    pltpu.sync_copy(th_hbm.at[pl.ds(base, RPW)], th_v)
    pltpu.sync_copy(xs_hbm, xs_v)
    pltpu.sync_copy(ys_hbm, ys_v)
    pltpu.sync_copy(zs_hbm, zs_v)
    zeros16i = jnp.zeros((16,), jnp.int32)
    for j in range(10):
        eqi_v[pl.ds(j * 16, 16)] = zeros16i

    lane = jax.lax.broadcasted_iota(jnp.int32, (16,), 0)

    def row_body(r, _):
        row = base + r
        pltpu.sync_copy(corr_hbm.at[row], row_v)
        t16 = plsc.load_gather(th_v, [jnp.full((16,), r, jnp.int32)])

        def chunk_body(c, carry):
            cgt, ceq = carry
            v = row_v[pl.ds(c * 16, 16)]
            mge = v >= t16

            def full(_):
                colv = lane + c * 16
                mgt = v > t16
                csg = plsc.cumsum(mgt.astype(jnp.int32))
                dstg = cgt + csg - 1
                plsc.store_scatter(gtv_v, [dstg], v, mask=mgt)
                plsc.store_scatter(gti_v, [dstg], colv, mask=mgt)
                meq = mge & ~mgt
                cse = plsc.cumsum(meq.astype(jnp.int32))
                meq2 = meq & (ceq + cse <= K)
                dste = ceq + cse - 1
                plsc.store_scatter(eqv_v, [dste], v, mask=meq2)
                plsc.store_scatter(eqi_v, [dste], colv, mask=meq2)
                return (cgt + jnp.sum(mgt.astype(jnp.int32)),
                        ceq + jnp.sum(meq2.astype(jnp.int32)))

            def skip(_):
                return (cgt, ceq)

            return lax.cond(jnp.any(mge), full, skip, 0)

        cgt, ceq = lax.fori_loop(0, NCH, chunk_body,
                                 (jnp.int32(0), jnp.int32(0)))
        for j in range(8):
            gtv_v[pl.ds(cgt + j * 16, 16)] = eqv_v[pl.ds(j * 16, 16)]
            gti_v[pl.ds(cgt + j * 16, 16)] = eqi_v[pl.ds(j * 16, 16)]
        for j in range(8):
            iv = gti_v[pl.ds(j * 16, 16)]
            gxb_v[pl.ds(j * 16, 16)] = plsc.load_gather(xs_v, [iv])
            gyb_v[pl.ds(j * 16, 16)] = plsc.load_gather(ys_v, [iv])
            gzb_v[pl.ds(j * 16, 16)] = plsc.load_gather(zs_v, [iv])
        pltpu.sync_copy(gtv_v.at[pl.ds(0, K)], vals_hbm.at[row])
        pltpu.sync_copy(gxb_v, gx_hbm.at[row])
        pltpu.sync_copy(gyb_v, gy_hbm.at[row])
        pltpu.sync_copy(gzb_v, gz_hbm.at[row])
        return 0

    lax.fori_loop(0, RPW, row_body, 0)


def _s2(corr, th, xs, ys, zs):
    mesh = plsc.VectorSubcoreMesh(core_axis_name="c", subcore_axis_name="s")
    fn = functools.partial(
        pl.kernel,
        mesh=mesh,
        compiler_params=pltpu.CompilerParams(needs_layout_passes=False),
        out_type=[
            jax.ShapeDtypeStruct((N, K), jnp.float32),
            jax.ShapeDtypeStruct((N, K), jnp.float32),
            jax.ShapeDtypeStruct((N, K), jnp.float32),
            jax.ShapeDtypeStruct((N, K), jnp.float32),
        ],
        scratch_types=[
            pltpu.VMEM((N,), jnp.float32),
            pltpu.VMEM((RPW,), jnp.float32),
            pltpu.VMEM((N,), jnp.float32),
            pltpu.VMEM((N,), jnp.float32),
            pltpu.VMEM((N,), jnp.float32),
            pltpu.VMEM((320,), jnp.float32),
            pltpu.VMEM((320,), jnp.int32),
            pltpu.VMEM((160,), jnp.float32),
            pltpu.VMEM((160,), jnp.int32),
            pltpu.VMEM((K,), jnp.float32),
            pltpu.VMEM((K,), jnp.float32),
            pltpu.VMEM((K,), jnp.float32),
        ],
    )(_s2_body)
    return fn(corr, th, xs, ys, zs)


def _s3_body(vals_ref, gx_ref, gy_ref, gz_ref, cx_ref, cy_ref, cz_ref,
             w1_ref, b1_ref, wk_sm, bk_sm,
             x1_ref, mx_ref, mn_ref, st_ref):
    vals = vals_ref[...]
    dx = gx_ref[...] - cx_ref[...]
    dy = gy_ref[...] - cy_ref[...]
    dz = gz_ref[...] - cz_ref[...]
    cols = []
    for lvl in range(NUM_LEVELS):
        inv_r = np.float32(1.0 / (BASE_SCALE * 2 ** lvl))
        dvx = jnp.round(dx * inv_r)
        dvy = jnp.round(dy * inv_r)
        dvz = jnp.round(dz * inv_r)
        valid = ((jnp.abs(dvx) <= 1.0) & (jnp.abs(dvy) <= 1.0)
                 & (jnp.abs(dvz) <= 1.0))
        cube = (dvx + 1.0) * 9.0 + (dvy + 1.0) * 3.0 + (dvz + 1.0)
        for c in range(27):
            mf = jnp.where(valid & (cube == np.float32(c)), 1.0, 0.0)
            ca = jnp.sum(vals * mf, axis=1, keepdims=True)
            cc = jnp.sum(mf, axis=1, keepdims=True)
            cols.append(ca / jnp.maximum(cc, 1.0))
    cols.append(jnp.zeros((BLK, 128 - 81), jnp.float32))
    feats = jnp.concatenate(cols, axis=1)
    x1 = lax.dot_general(feats, w1_ref[...], (((1,), (1,)), ((), ())),
                         preferred_element_type=jnp.float32) + b1_ref[...]
    x1_ref[...] = x1
    dist = dx * dx + dy * dy + dz * dz
    t97 = _kth_largest(dist, 97)
    mlt = dist < t97
    ones_col = jnp.ones((128, 1), jnp.float32)
    c_lt = lax.dot_general(jnp.where(mlt, 1.0, 0.0), ones_col,
                           (((1,), (0,)), ((), ())),
                           preferred_element_type=jnp.float32)
    meq = dist == t97
    corr_eq = jnp.where(meq, vals, -3.0e38)
    c_cut = _kth_largest(corr_eq, 32.0 - c_lt)
    mknn = mlt | (meq & (vals >= c_cut))
    mkf = jnp.where(mknn, 1.0, 0.0)
    f4 = (vals, dx, dy, dz)
    cnt = jnp.sum(mkf)
    sf = [jnp.sum(f * mkf) for f in f4]
    sff = [[jnp.sum(f4[i] * f4[j] * mkf) for j in range(4)] for i in range(4)]
    mx_ref[...] = jnp.zeros((BLK, 128), jnp.float32)
    mn_ref[...] = jnp.zeros((BLK, 128), jnp.float32)
    for c in range(64):
        u = (vals * wk_sm[c, 0] + dx * wk_sm[c, 1] + dy * wk_sm[c, 2]
             + dz * wk_sm[c, 3] + bk_sm[c])
        mx_ref[:, c:c + 1] = jnp.max(jnp.where(mknn, u, -3.0e38), axis=1,
                                     keepdims=True)
        mn_ref[:, c:c + 1] = jnp.min(jnp.where(mknn, u, 3.0e38), axis=1,
                                     keepdims=True)
    s1v = jnp.sum(x1, axis=0, keepdims=True)
    s2v = jnp.sum(x1 * x1, axis=0, keepdims=True)
    lane = lax.broadcasted_iota(jnp.int32, (1, 128), 1)
    knrow = jnp.zeros((1, 128), jnp.float32)
    for i in range(4):
        knrow = jnp.where(lane == i, sf[i], knrow)
    for i in range(4):
        for j in range(4):
            knrow = jnp.where(lane == 4 + 4 * i + j, sff[i][j], knrow)
    knrow = jnp.where(lane == 20, cnt, knrow)
    row = lax.broadcasted_iota(jnp.int32, (8, 128), 0)
    contrib = jnp.where(row == 0, s1v, jnp.where(row == 1, s2v,
                        jnp.where(row == 2, knrow, 0.0)))

    @pl.when(pl.program_id(0) == 0)
    def _():
        st_ref[...] = jnp.zeros((8, 128), jnp.float32)

    st_ref[...] += contrib


def _s3(vals, gx, gy, gz, cxb, cyb, czb, W1p, b1r, Wk, bk):
    grid = N // BLK
    bs = pl.BlockSpec((BLK, 128), lambda i: (i, 0))
    full = pl.BlockSpec((128, 128), lambda i: (0, 0))
    row1 = pl.BlockSpec((1, 128), lambda i: (0, 0))
    smem = pl.BlockSpec(memory_space=pltpu.SMEM)
    x1, mx, mn, st = pl.pallas_call(
        _s3_body,
        grid=(grid,),
        in_specs=[bs, bs, bs, bs, bs, bs, bs, full, row1, smem, smem],
        out_specs=[bs, bs, bs, pl.BlockSpec((8, 128), lambda i: (0, 0))],
        out_shape=[
            jax.ShapeDtypeStruct((N, 128), jnp.float32),
            jax.ShapeDtypeStruct((N, 128), jnp.float32),
            jax.ShapeDtypeStruct((N, 128), jnp.float32),
            jax.ShapeDtypeStruct((8, 128), jnp.float32),
        ],
    )(vals, gx, gy, gz, cxb, cyb, czb, W1p, b1r, Wk, bk)
    return x1, mx, mn, st


def _s4_body(x1_ref, mx_ref, mn_ref, st_ref, st_sm,
             g1_ref, be1_ref, p1_ref, w2_ref, b2_ref,
             wk_ref, bk_ref, gk_ref, bek_ref, pk_ref, wo_ref, bo_ref,
             out_ref):
    li = lax.broadcasted_iota(jnp.int32, (128, 128), 0)
    lj = lax.broadcasted_iota(jnp.int32, (128, 128), 1)
    lane = lax.broadcasted_iota(jnp.int32, (1, 128), 1)
    s1 = st_ref[0:1, :]
    s2 = st_ref[1:2, :]
    g16 = jnp.where((li // 16) == (lj // 16), 1.0, 0.0)
    gs1 = lax.dot_general(s1, g16, (((1,), (0,)), ((), ())),
                          preferred_element_type=jnp.float32)
    gs2 = lax.dot_general(s2, g16, (((1,), (0,)), ((), ())),
                          preferred_element_type=jnp.float32)
    denom = np.float32(16 * N)
    mean = gs1 / denom
    var = gs2 / denom - mean * mean
    a = g1_ref[...] * lax.rsqrt(var + 1e-5)
    d = be1_ref[...] - mean * a
    xh = x1_ref[...] * a + d
    xp = jnp.where(xh > 0, xh, p1_ref[...] * xh)
    vox = lax.dot_general(xp, w2_ref[...], (((1,), (1,)), ((), ())),
                          preferred_element_type=jnp.float32) + b2_ref[...]
    cnt = st_sm[2, 20]
    fsum = jnp.zeros((1, 128), jnp.float32)
    for i in range(4):
        fsum = jnp.where(lane == i, st_sm[2, i], fsum)
    sffm = jnp.zeros((128, 128), jnp.float32)
    for i in range(4):
        for j in range(4):
            sffm = jnp.where((li == i) & (lj == j), st_sm[2, 4 + 4 * i + j],
                             sffm)
    wk = wk_ref[...]
    bk = bk_ref[...]
    s1k_lin = lax.dot_general(fsum, wk, (((1,), (1,)), ((), ())),
                              preferred_element_type=jnp.float32)
    s1k = s1k_lin + cnt * bk
    t1 = lax.dot_general(wk, sffm, (((1,), (0,)), ((), ())),
                         preferred_element_type=jnp.float32)
    ones_row = jnp.ones((1, 128), jnp.float32)
    quad = lax.dot_general(ones_row, t1 * wk, (((1,), (1,)), ((), ())),
                           preferred_element_type=jnp.float32)
    s2k = quad + 2.0 * bk * s1k_lin + cnt * bk * bk
    g8 = jnp.where((li // 8) == (lj // 8), 1.0, 0.0)
    gk1 = lax.dot_general(s1k, g8, (((1,), (0,)), ((), ())),
                          preferred_element_type=jnp.float32)
    gk2 = lax.dot_general(s2k, g8, (((1,), (0,)), ((), ())),
                          preferred_element_type=jnp.float32)
    cdenom = 8.0 * cnt
    meank = gk1 / cdenom
    vark = gk2 / cdenom - meank * meank
    ak = gk_ref[...] * lax.rsqrt(vark + 1e-5)
    dk = bek_ref[...] - meank * ak
    z = jnp.where(ak > 0, mx_ref[...], mn_ref[...])
    zz = z * ak + dk
    zp = jnp.where(zz > 0, zz, pk_ref[...] * zz)
    knn = lax.dot_general(zp, wo_ref[...], (((1,), (1,)), ((), ())),
                          preferred_element_type=jnp.float32) + bo_ref[...]
    out_ref[...] = vox + knn


def _s4(x1, mx, mn, st, g1r, be1r, p1r, W2p, b2p, Wkp, bkp, gkp, bekp, pkp,
        Wop, bop):
    grid = N // BLK
    bs = pl.BlockSpec((BLK, 128), lambda i: (i, 0))
    full = pl.BlockSpec((128, 128), lambda i: (0, 0))
    row1 = pl.BlockSpec((1, 128), lambda i: (0, 0))
    st8 = pl.BlockSpec((8, 128), lambda i: (0, 0))
    smem = pl.BlockSpec(memory_space=pltpu.SMEM)
    out = pl.pallas_call(
        _s4_body,
        grid=(grid,),
        in_specs=[bs, bs, bs, st8, smem,
                  row1, row1, row1, full, row1,
                  full, row1, row1, row1, row1, full, row1],
        out_specs=bs,
        out_shape=jax.ShapeDtypeStruct((N, 128), jnp.float32),
    )(x1, mx, mn, st, st,
      g1r, be1r, p1r, W2p, b2p,
      Wkp, bkp, gkp, bekp, pkp, Wop, bop)
    return out


def _pad_rc(w, rows, cols):
    return jnp.zeros((rows, cols), w.dtype).at[:w.shape[0], :w.shape[1]].set(w)


def _pad_row(v, cols):
    return jnp.zeros((1, cols), v.dtype).at[0, :v.shape[0]].set(v)


def kernel(fmap1, fmap2, xyz2, coords, W1, b1, g1, be1, p1, W2, b2, Wk, bk, gk, bek, pk, Wo, bo):
    f1 = fmap1[0]
    f2 = fmap2[0]
    corr2d, th = _s1(f1, f2)
    xs = xyz2[0, :, 0]
    ys = xyz2[0, :, 1]
    zs = xyz2[0, :, 2]
    vals, gx, gy, gz = _s2(corr2d, th, xs, ys, zs)
    cxb = jnp.broadcast_to(coords[0, :, 0:1], (N, 128))
    cyb = jnp.broadcast_to(coords[0, :, 1:2], (N, 128))
    czb = jnp.broadcast_to(coords[0, :, 2:3], (N, 128))
    W1p = _pad_rc(W1, 128, 128)
    b1r = b1.reshape(1, 128)
    x1, mx, mn, st = _s3(vals, gx, gy, gz, cxb, cyb, czb, W1p, b1r, Wk, bk)
    out2d = _s4(
        x1, mx, mn, st,
        g1.reshape(1, 128), be1.reshape(1, 128),
        jnp.broadcast_to(p1.reshape(1, 1), (1, 128)),
        _pad_rc(W2, 128, 128), _pad_row(b2, 128),
        _pad_rc(Wk, 128, 128), _pad_row(bk, 128),
        _pad_row(gk, 128), _pad_row(bek, 128),
        jnp.broadcast_to(pk.reshape(1, 1), (1, 128)),
        _pad_rc(Wo, 128, 128), _pad_row(bo, 128),
    )
    return out2d[:, :64].T[None]


# speedup vs baseline: 1.2977x; 1.2977x over previous
import functools

import jax
import jax.numpy as jnp
import numpy as np
from jax import lax
from jax.experimental import pallas as pl
from jax.experimental.pallas import tpu as pltpu
from jax.experimental.pallas import tpu_sc as plsc

NUM_LEVELS = 3
BASE_SCALE = 0.25
RESOLUTION = 3
TRUNCATE_K = 128
KNN = 32

N = 8192
DIM = 128
K = TRUNCATE_K
BLK = 256
MIN32 = np.int32(-2147483648)

NW = 32
RPW = N // NW
NCH = N // 16


def _kth_largest(x, kth):
    bits = lax.bitcast_convert_type(x, jnp.int32)
    key = jnp.where(bits >= 0, bits, ~bits ^ MIN32)
    ones = jnp.ones((x.shape[1], 1), jnp.float32)
    t_u = jnp.zeros((x.shape[0], 1), jnp.int32)
    for bit in range(31, -1, -1):
        m = np.uint32(1 << bit).astype(np.int32)
        cand = t_u | m
        ind = (key >= (cand ^ MIN32)).astype(jnp.float32)
        cnt = lax.dot_general(ind, ones, (((1,), (0,)), ((), ())),
                              preferred_element_type=jnp.float32)
        t_u = jnp.where(cnt >= kth, cand, t_u)
    fbits = jnp.where(t_u < 0, t_u ^ MIN32, ~t_u)
    return lax.bitcast_convert_type(fbits, jnp.float32)


def _s1_body(a_ref, b_ref, corr_ref, th_ref):
    a = a_ref[...]
    b = b_ref[...]
    corr = lax.dot_general(a, b, (((0,), (0,)), ((), ())),
                           preferred_element_type=jnp.float32)
    corr = corr * np.float32(1.0 / np.sqrt(DIM))
    corr_ref[...] = corr
    t = _kth_largest(corr, K)
    th_ref[...] = t.reshape(1, 1, BLK)


def _s1(f1, f2):
    grid = N // BLK
    corr, th = pl.pallas_call(
        _s1_body,
        grid=(grid,),
        in_specs=[
            pl.BlockSpec((DIM, BLK), lambda i: (0, i)),
            pl.BlockSpec((DIM, N), lambda i: (0, 0)),
        ],
        out_specs=[
            pl.BlockSpec((BLK, N), lambda i: (i, 0)),
            pl.BlockSpec((1, 1, BLK), lambda i: (i, 0, 0)),
        ],
        out_shape=[
            jax.ShapeDtypeStruct((N, N), jnp.float32),
            jax.ShapeDtypeStruct((grid, 1, BLK), jnp.float32),
        ],
    )(f1, f2)
    return corr, th.reshape(N)


def _s2_body(corr_hbm, th_hbm, xs_hbm, ys_hbm, zs_hbm,
             vals_hbm, gx_hbm, gy_hbm, gz_hbm,
             row_v, th_v, xs_v, ys_v, zs_v,
             gtv_v, gti_v, eqv_v, eqi_v, gxb_v, gyb_v, gzb_v):
    wid = lax.axis_index("s") * 2 + lax.axis_index("c")
    base = wid * RPW
    pltpu.sync_copy(th_hbm.at[pl.ds(base, RPW)], th_v)
    pltpu.sync_copy(xs_hbm, xs_v)
    pltpu.sync_copy(ys_hbm, ys_v)
    pltpu.sync_copy(zs_hbm, zs_v)
    zeros16i = jnp.zeros((16,), jnp.int32)
    for j in range(10):
        eqi_v[pl.ds(j * 16, 16)] = zeros16i

    lane = jax.lax.broadcasted_iota(jnp.int32, (16,), 0)

    def row_body(r, _):
        row = base + r
        pltpu.sync_copy(corr_hbm.at[row], row_v)
        t16 = plsc.load_gather(th_v, [jnp.full((16,), r, jnp.int32)])

        def chunk_body(c, carry):
            cgt, ceq = carry
            v = row_v[pl.ds(c * 16, 16)]
            colv = lane + c * 16
            mgt = v > t16
            csg = plsc.cumsum(mgt.astype(jnp.int32))
            dstg = cgt + csg - 1
            plsc.store_scatter(gtv_v, [dstg], v, mask=mgt)
            plsc.store_scatter(gti_v, [dstg], colv, mask=mgt)
            cgt = cgt + jnp.sum(mgt.astype(jnp.int32))
            meq = v == t16
            cse = plsc.cumsum(meq.astype(jnp.int32))
            meq = meq & (ceq + cse <= K)
            dste = ceq + cse - 1
            plsc.store_scatter(eqv_v, [dste], v, mask=meq)
            plsc.store_scatter(eqi_v, [dste], colv, mask=meq)
            ceq = ceq + jnp.sum(meq.astype(jnp.int32))
            return (cgt, ceq)

        cgt, ceq = lax.fori_loop(0, NCH, chunk_body,
                                 (jnp.int32(0), jnp.int32(0)))
        for j in range(8):
            gtv_v[pl.ds(cgt + j * 16, 16)] = eqv_v[pl.ds(j * 16, 16)]
            gti_v[pl.ds(cgt + j * 16, 16)] = eqi_v[pl.ds(j * 16, 16)]
        for j in range(8):
            iv = gti_v[pl.ds(j * 16, 16)]
            gxb_v[pl.ds(j * 16, 16)] = plsc.load_gather(xs_v, [iv])
            gyb_v[pl.ds(j * 16, 16)] = plsc.load_gather(ys_v, [iv])
            gzb_v[pl.ds(j * 16, 16)] = plsc.load_gather(zs_v, [iv])
        pltpu.sync_copy(gtv_v.at[pl.ds(0, K)], vals_hbm.at[row])
        pltpu.sync_copy(gxb_v, gx_hbm.at[row])
        pltpu.sync_copy(gyb_v, gy_hbm.at[row])
        pltpu.sync_copy(gzb_v, gz_hbm.at[row])
        return 0

    lax.fori_loop(0, RPW, row_body, 0)


def _s2(corr, th, xs, ys, zs):
    mesh = plsc.VectorSubcoreMesh(core_axis_name="c", subcore_axis_name="s")
    fn = functools.partial(
        pl.kernel,
        mesh=mesh,
        compiler_params=pltpu.CompilerParams(needs_layout_passes=False),
        out_type=[
            jax.ShapeDtypeStruct((N, K), jnp.float32),
            jax.ShapeDtypeStruct((N, K), jnp.float32),
            jax.ShapeDtypeStruct((N, K), jnp.float32),
            jax.ShapeDtypeStruct((N, K), jnp.float32),
        ],
        scratch_types=[
            pltpu.VMEM((N,), jnp.float32),
            pltpu.VMEM((RPW,), jnp.float32),
            pltpu.VMEM((N,), jnp.float32),
            pltpu.VMEM((N,), jnp.float32),
            pltpu.VMEM((N,), jnp.float32),
            pltpu.VMEM((320,), jnp.float32),
            pltpu.VMEM((320,), jnp.int32),
            pltpu.VMEM((160,), jnp.float32),
            pltpu.VMEM((160,), jnp.int32),
            pltpu.VMEM((K,), jnp.float32),
            pltpu.VMEM((K,), jnp.float32),
            pltpu.VMEM((K,), jnp.float32),
        ],
    )(_s2_body)
    return fn(corr, th, xs, ys, zs)


def _s3_body(vals_ref, gx_ref, gy_ref, gz_ref, cx_ref, cy_ref, cz_ref,
             w1_ref, b1_ref, wk_sm, bk_sm,
             x1_ref, mx_ref, mn_ref, st_ref):
    vals = vals_ref[...]
    dx = gx_ref[...] - cx_ref[...]
    dy = gy_ref[...] - cy_ref[...]
    dz = gz_ref[...] - cz_ref[...]
    cols = []
    for lvl in range(NUM_LEVELS):
        inv_r = np.float32(1.0 / (BASE_SCALE * 2 ** lvl))
        dvx = jnp.round(dx * inv_r)
        dvy = jnp.round(dy * inv_r)
        dvz = jnp.round(dz * inv_r)
        valid = ((jnp.abs(dvx) <= 1.0) & (jnp.abs(dvy) <= 1.0)
                 & (jnp.abs(dvz) <= 1.0))
        cube = (dvx + 1.0) * 9.0 + (dvy + 1.0) * 3.0 + (dvz + 1.0)
        for c in range(27):
            mf = jnp.where(valid & (cube == np.float32(c)), 1.0, 0.0)
            ca = jnp.sum(vals * mf, axis=1, keepdims=True)
            cc = jnp.sum(mf, axis=1, keepdims=True)
            cols.append(ca / jnp.maximum(cc, 1.0))
    cols.append(jnp.zeros((BLK, 128 - 81), jnp.float32))
    feats = jnp.concatenate(cols, axis=1)
    x1 = lax.dot_general(feats, w1_ref[...], (((1,), (1,)), ((), ())),
                         preferred_element_type=jnp.float32) + b1_ref[...]
    x1_ref[...] = x1
    dist = dx * dx + dy * dy + dz * dz
    t97 = _kth_largest(dist, 97)
    mlt = dist < t97
    ones_col = jnp.ones((128, 1), jnp.float32)
    c_lt = lax.dot_general(jnp.where(mlt, 1.0, 0.0), ones_col,
                           (((1,), (0,)), ((), ())),
                           preferred_element_type=jnp.float32)
    meq = dist == t97
    corr_eq = jnp.where(meq, vals, -3.0e38)
    c_cut = _kth_largest(corr_eq, 32.0 - c_lt)
    mknn = mlt | (meq & (vals >= c_cut))
    mkf = jnp.where(mknn, 1.0, 0.0)
    f4 = (vals, dx, dy, dz)
    cnt = jnp.sum(mkf)
    sf = [jnp.sum(f * mkf) for f in f4]
    sff = [[jnp.sum(f4[i] * f4[j] * mkf) for j in range(4)] for i in range(4)]
    mx_ref[...] = jnp.zeros((BLK, 128), jnp.float32)
    mn_ref[...] = jnp.zeros((BLK, 128), jnp.float32)
    for c in range(64):
        u = (vals * wk_sm[c, 0] + dx * wk_sm[c, 1] + dy * wk_sm[c, 2]
             + dz * wk_sm[c, 3] + bk_sm[c])
        mx_ref[:, c:c + 1] = jnp.max(jnp.where(mknn, u, -3.0e38), axis=1,
                                     keepdims=True)
        mn_ref[:, c:c + 1] = jnp.min(jnp.where(mknn, u, 3.0e38), axis=1,
                                     keepdims=True)
    s1v = jnp.sum(x1, axis=0, keepdims=True)
    s2v = jnp.sum(x1 * x1, axis=0, keepdims=True)
    lane = lax.broadcasted_iota(jnp.int32, (1, 128), 1)
    knrow = jnp.zeros((1, 128), jnp.float32)
    for i in range(4):
        knrow = jnp.where(lane == i, sf[i], knrow)
    for i in range(4):
        for j in range(4):
            knrow = jnp.where(lane == 4 + 4 * i + j, sff[i][j], knrow)
    knrow = jnp.where(lane == 20, cnt, knrow)
    row = lax.broadcasted_iota(jnp.int32, (8, 128), 0)
    contrib = jnp.where(row == 0, s1v, jnp.where(row == 1, s2v,
                        jnp.where(row == 2, knrow, 0.0)))

    @pl.when(pl.program_id(0) == 0)
    def _():
        st_ref[...] = jnp.zeros((8, 128), jnp.float32)

    st_ref[...] += contrib


def _s3(vals, gx, gy, gz, cxb, cyb, czb, W1p, b1r, Wk, bk):
    grid = N // BLK
    bs = pl.BlockSpec((BLK, 128), lambda i: (i, 0))
    full = pl.BlockSpec((128, 128), lambda i: (0, 0))
    row1 = pl.BlockSpec((1, 128), lambda i: (0, 0))
    smem = pl.BlockSpec(memory_space=pltpu.SMEM)
    x1, mx, mn, st = pl.pallas_call(
        _s3_body,
        grid=(grid,),
        in_specs=[bs, bs, bs, bs, bs, bs, bs, full, row1, smem, smem],
        out_specs=[bs, bs, bs, pl.BlockSpec((8, 128), lambda i: (0, 0))],
        out_shape=[
            jax.ShapeDtypeStruct((N, 128), jnp.float32),
            jax.ShapeDtypeStruct((N, 128), jnp.float32),
            jax.ShapeDtypeStruct((N, 128), jnp.float32),
            jax.ShapeDtypeStruct((8, 128), jnp.float32),
        ],
    )(vals, gx, gy, gz, cxb, cyb, czb, W1p, b1r, Wk, bk)
    return x1, mx, mn, st


def _s4_body(x1_ref, mx_ref, mn_ref, st_ref, st_sm,
             g1_ref, be1_ref, p1_ref, w2_ref, b2_ref,
             wk_ref, bk_ref, gk_ref, bek_ref, pk_ref, wo_ref, bo_ref,
             out_ref):
    li = lax.broadcasted_iota(jnp.int32, (128, 128), 0)
    lj = lax.broadcasted_iota(jnp.int32, (128, 128), 1)
    lane = lax.broadcasted_iota(jnp.int32, (1, 128), 1)
    s1 = st_ref[0:1, :]
    s2 = st_ref[1:2, :]
    g16 = jnp.where((li // 16) == (lj // 16), 1.0, 0.0)
    gs1 = lax.dot_general(s1, g16, (((1,), (0,)), ((), ())),
                          preferred_element_type=jnp.float32)
    gs2 = lax.dot_general(s2, g16, (((1,), (0,)), ((), ())),
                          preferred_element_type=jnp.float32)
    denom = np.float32(16 * N)
    mean = gs1 / denom
    var = gs2 / denom - mean * mean
    a = g1_ref[...] * lax.rsqrt(var + 1e-5)
    d = be1_ref[...] - mean * a
    xh = x1_ref[...] * a + d
    xp = jnp.where(xh > 0, xh, p1_ref[...] * xh)
    vox = lax.dot_general(xp, w2_ref[...], (((1,), (1,)), ((), ())),
                          preferred_element_type=jnp.float32) + b2_ref[...]
    cnt = st_sm[2, 20]
    fsum = jnp.zeros((1, 128), jnp.float32)
    for i in range(4):
        fsum = jnp.where(lane == i, st_sm[2, i], fsum)
    sffm = jnp.zeros((128, 128), jnp.float32)
    for i in range(4):
        for j in range(4):
            sffm = jnp.where((li == i) & (lj == j), st_sm[2, 4 + 4 * i + j],
                             sffm)
    wk = wk_ref[...]
    bk = bk_ref[...]
    s1k_lin = lax.dot_general(fsum, wk, (((1,), (1,)), ((), ())),
                              preferred_element_type=jnp.float32)
    s1k = s1k_lin + cnt * bk
    t1 = lax.dot_general(wk, sffm, (((1,), (0,)), ((), ())),
                         preferred_element_type=jnp.float32)
    ones_row = jnp.ones((1, 128), jnp.float32)
    quad = lax.dot_general(ones_row, t1 * wk, (((1,), (1,)), ((), ())),
                           preferred_element_type=jnp.float32)
    s2k = quad + 2.0 * bk * s1k_lin + cnt * bk * bk
    g8 = jnp.where((li // 8) == (lj // 8), 1.0, 0.0)
    gk1 = lax.dot_general(s1k, g8, (((1,), (0,)), ((), ())),
                          preferred_element_type=jnp.float32)
    gk2 = lax.dot_general(s2k, g8, (((1,), (0,)), ((), ())),
                          preferred_element_type=jnp.float32)
    cdenom = 8.0 * cnt
    meank = gk1 / cdenom
    vark = gk2 / cdenom - meank * meank
    ak = gk_ref[...] * lax.rsqrt(vark + 1e-5)
    dk = bek_ref[...] - meank * ak
    z = jnp.where(ak > 0, mx_ref[...], mn_ref[...])
    zz = z * ak + dk
    zp = jnp.where(zz > 0, zz, pk_ref[...] * zz)
    knn = lax.dot_general(zp, wo_ref[...], (((1,), (1,)), ((), ())),
                          preferred_element_type=jnp.float32) + bo_ref[...]
    out_ref[...] = vox + knn


def _s4(x1, mx, mn, st, g1r, be1r, p1r, W2p, b2p, Wkp, bkp, gkp, bekp, pkp,
        Wop, bop):
    grid = N // BLK
    bs = pl.BlockSpec((BLK, 128), lambda i: (i, 0))
    full = pl.BlockSpec((128, 128), lambda i: (0, 0))
    row1 = pl.BlockSpec((1, 128), lambda i: (0, 0))
    st8 = pl.BlockSpec((8, 128), lambda i: (0, 0))
    smem = pl.BlockSpec(memory_space=pltpu.SMEM)
    out = pl.pallas_call(
        _s4_body,
        grid=(grid,),
        in_specs=[bs, bs, bs, st8, smem,
                  row1, row1, row1, full, row1,
                  full, row1, row1, row1, row1, full, row1],
        out_specs=bs,
        out_shape=jax.ShapeDtypeStruct((N, 128), jnp.float32),
    )(x1, mx, mn, st, st,
      g1r, be1r, p1r, W2p, b2p,
      Wkp, bkp, gkp, bekp, pkp, Wop, bop)
    return out


def _pad_rc(w, rows, cols):
    return jnp.zeros((rows, cols), w.dtype).at[:w.shape[0], :w.shape[1]].set(w)


def _pad_row(v, cols):
    return jnp.zeros((1, cols), v.dtype).at[0, :v.shape[0]].set(v)


def kernel(fmap1, fmap2, xyz2, coords, W1, b1, g1, be1, p1, W2, b2, Wk, bk, gk, bek, pk, Wo, bo):
    f1 = fmap1[0]
    f2 = fmap2[0]
    corr2d, th = _s1(f1, f2)
    xs = xyz2[0, :, 0]
    ys = xyz2[0, :, 1]
    zs = xyz2[0, :, 2]
    vals, gx, gy, gz = _s2(corr2d, th, xs, ys, zs)
    cxb = jnp.broadcast_to(coords[0, :, 0:1], (N, 128))
    cyb = jnp.broadcast_to(coords[0, :, 1:2], (N, 128))
    czb = jnp.broadcast_to(coords[0, :, 2:3], (N, 128))
    W1p = _pad_rc(W1, 128, 128)
    b1r = b1.reshape(1, 128)
    x1, mx, mn, st = _s3(vals, gx, gy, gz, cxb, cyb, czb, W1p, b1r, Wk, bk)
    out2d = _s4(
        x1, mx, mn, st,
        g1.reshape(1, 128), be1.reshape(1, 128),
        jnp.broadcast_to(p1.reshape(1, 1), (1, 128)),
        _pad_rc(W2, 128, 128), _pad_row(b2, 128),
        _pad_rc(Wk, 128, 128), _pad_row(bk, 128),
        _pad_row(gk, 128), _pad_row(bek, 128),
        jnp.broadcast_to(pk.reshape(1, 1), (1, 128)),
        _pad_rc(Wo, 128, 128), _pad_row(bo, 128),
    )
    return out2d[:, :64].T[None]


# speedup vs baseline: 1.3656x; 1.0523x over previous
import functools

import jax
import jax.numpy as jnp
import numpy as np
from jax import lax
from jax.experimental import pallas as pl
from jax.experimental.pallas import tpu as pltpu
from jax.experimental.pallas import tpu_sc as plsc

NUM_LEVELS = 3
BASE_SCALE = 0.25
RESOLUTION = 3
TRUNCATE_K = 128
KNN = 32

N = 8192
DIM = 128
K = TRUNCATE_K
BLK = 256
MIN32 = np.int32(-2147483648)

NW = 32
RPW = N // NW
NCH = N // 16


def _kth_largest(x, kth):
    bits = lax.bitcast_convert_type(x, jnp.int32)
    key = jnp.where(bits >= 0, bits, ~bits ^ MIN32)
    ones = jnp.ones((x.shape[1], 1), jnp.float32)
    t_u = jnp.zeros((x.shape[0], 1), jnp.int32)
    for bit in range(31, -1, -1):
        m = np.uint32(1 << bit).astype(np.int32)
        cand = t_u | m
        ind = (key >= (cand ^ MIN32)).astype(jnp.float32)
        cnt = lax.dot_general(ind, ones, (((1,), (0,)), ((), ())),
                              preferred_element_type=jnp.float32)
        t_u = jnp.where(cnt >= kth, cand, t_u)
    fbits = jnp.where(t_u < 0, t_u ^ MIN32, ~t_u)
    return lax.bitcast_convert_type(fbits, jnp.float32)


def _s1_body(a_ref, b_ref, corr_ref, th_ref):
    a = a_ref[...]
    b = b_ref[...]
    corr = lax.dot_general(a, b, (((0,), (0,)), ((), ())),
                           preferred_element_type=jnp.float32)
    corr = corr * np.float32(1.0 / np.sqrt(DIM))
    corr_ref[...] = corr
    t = _kth_largest(corr, K)
    th_ref[...] = t.reshape(1, 1, BLK)


def _s1(f1, f2):
    grid = N // BLK
    corr, th = pl.pallas_call(
        _s1_body,
        grid=(grid,),
        in_specs=[
            pl.BlockSpec((DIM, BLK), lambda i: (0, i)),
            pl.BlockSpec((DIM, N), lambda i: (0, 0)),
        ],
        out_specs=[
            pl.BlockSpec((BLK, N), lambda i: (i, 0)),
            pl.BlockSpec((1, 1, BLK), lambda i: (i, 0, 0)),
        ],
        out_shape=[
            jax.ShapeDtypeStruct((N, N), jnp.float32),
            jax.ShapeDtypeStruct((grid, 1, BLK), jnp.float32),
        ],
    )(f1, f2)
    return corr, th.reshape(N)


def _s2_body(corr_hbm, th_hbm, xs_hbm, ys_hbm, zs_hbm,
             vals_hbm, gx_hbm, gy_hbm, gz_hbm,
             row_v, th_v, xs_v, ys_v, zs_v,
             gtv_v, gti_v, eqv_v, eqi_v, gxb_v, gyb_v, gzb_v):
    wid = lax.axis_index("s") * 2 + lax.axis_index("c")
    base = wid * RPW
    pltpu.sync_copy(th_hbm.at[pl.ds(base, RPW)], th_v)
    pltpu.sync_copy(xs_hbm, xs_v)
    pltpu.sync_copy(ys_hbm, ys_v)
    pltpu.sync_copy(zs_hbm, zs_v)
    zeros16i = jnp.zeros((16,), jnp.int32)
    for j in range(10):
        eqi_v[pl.ds(j * 16, 16)] = zeros16i

    lane = jax.lax.broadcasted_iota(jnp.int32, (16,), 0)

    def row_body(r, _):
        row = base + r
        pltpu.sync_copy(corr_hbm.at[row], row_v)
        t16 = plsc.load_gather(th_v, [jnp.full((16,), r, jnp.int32)])

        def chunk_body(c4, carry):
            cgt_v, ceq_v = carry
            for u in range(4):
                c = c4 * 4 + u
                v = row_v[pl.ds(c * 16, 16)]
                colv = lane + c * 16
                mgt = v > t16
                csg = plsc.cumsum(mgt.astype(jnp.int32))
                dstg = cgt_v + csg - 1
                plsc.store_scatter(gtv_v, [dstg], v, mask=mgt)
                plsc.store_scatter(gti_v, [dstg], colv, mask=mgt)
                cgt_v = cgt_v + plsc.all_reduce_population_count(mgt)
                meq0 = v == t16
                cse = plsc.cumsum(meq0.astype(jnp.int32))
                meq = meq0 & (ceq_v + cse <= K)
                dste = ceq_v + cse - 1
                plsc.store_scatter(eqv_v, [dste], v, mask=meq)
                plsc.store_scatter(eqi_v, [dste], colv, mask=meq)
                ceq_v = ceq_v + jnp.minimum(
                    plsc.all_reduce_population_count(meq0),
                    jnp.maximum(K - ceq_v, 0))
            return (cgt_v, ceq_v)

        z16 = jnp.zeros((16,), jnp.int32)
        cgt_v, _ceq_v = lax.fori_loop(0, NCH // 4, chunk_body, (z16, z16))
        for j in range(8):
            dst = cgt_v + lane + j * 16
            plsc.store_scatter(gtv_v, [dst], eqv_v[pl.ds(j * 16, 16)])
            plsc.store_scatter(gti_v, [dst], eqi_v[pl.ds(j * 16, 16)])
        for j in range(8):
            iv = gti_v[pl.ds(j * 16, 16)]
            gxb_v[pl.ds(j * 16, 16)] = plsc.load_gather(xs_v, [iv])
            gyb_v[pl.ds(j * 16, 16)] = plsc.load_gather(ys_v, [iv])
            gzb_v[pl.ds(j * 16, 16)] = plsc.load_gather(zs_v, [iv])
        pltpu.sync_copy(gtv_v.at[pl.ds(0, K)], vals_hbm.at[row])
        pltpu.sync_copy(gxb_v, gx_hbm.at[row])
        pltpu.sync_copy(gyb_v, gy_hbm.at[row])
        pltpu.sync_copy(gzb_v, gz_hbm.at[row])
        return 0

    lax.fori_loop(0, RPW, row_body, 0)


def _s2(corr, th, xs, ys, zs):
    mesh = plsc.VectorSubcoreMesh(core_axis_name="c", subcore_axis_name="s")
    fn = functools.partial(
        pl.kernel,
        mesh=mesh,
        compiler_params=pltpu.CompilerParams(needs_layout_passes=False),
        out_type=[
            jax.ShapeDtypeStruct((N, K), jnp.float32),
            jax.ShapeDtypeStruct((N, K), jnp.float32),
            jax.ShapeDtypeStruct((N, K), jnp.float32),
            jax.ShapeDtypeStruct((N, K), jnp.float32),
        ],
        scratch_types=[
            pltpu.VMEM((N,), jnp.float32),
            pltpu.VMEM((RPW,), jnp.float32),
            pltpu.VMEM((N,), jnp.float32),
            pltpu.VMEM((N,), jnp.float32),
            pltpu.VMEM((N,), jnp.float32),
            pltpu.VMEM((320,), jnp.float32),
            pltpu.VMEM((320,), jnp.int32),
            pltpu.VMEM((160,), jnp.float32),
            pltpu.VMEM((160,), jnp.int32),
            pltpu.VMEM((K,), jnp.float32),
            pltpu.VMEM((K,), jnp.float32),
            pltpu.VMEM((K,), jnp.float32),
        ],
    )(_s2_body)
    return fn(corr, th, xs, ys, zs)


def _s3_body(vals_ref, gx_ref, gy_ref, gz_ref, cx_ref, cy_ref, cz_ref,
             w1_ref, b1_ref, wk_sm, bk_sm,
             x1_ref, mx_ref, mn_ref, st_ref):
    vals = vals_ref[...]
    dx = gx_ref[...] - cx_ref[...]
    dy = gy_ref[...] - cy_ref[...]
    dz = gz_ref[...] - cz_ref[...]
    cols = []
    for lvl in range(NUM_LEVELS):
        inv_r = np.float32(1.0 / (BASE_SCALE * 2 ** lvl))
        dvx = jnp.round(dx * inv_r)
        dvy = jnp.round(dy * inv_r)
        dvz = jnp.round(dz * inv_r)
        valid = ((jnp.abs(dvx) <= 1.0) & (jnp.abs(dvy) <= 1.0)
                 & (jnp.abs(dvz) <= 1.0))
        cube = (dvx + 1.0) * 9.0 + (dvy + 1.0) * 3.0 + (dvz + 1.0)
        for c in range(27):
            mf = jnp.where(valid & (cube == np.float32(c)), 1.0, 0.0)
            ca = jnp.sum(vals * mf, axis=1, keepdims=True)
            cc = jnp.sum(mf, axis=1, keepdims=True)
            cols.append(ca / jnp.maximum(cc, 1.0))
    cols.append(jnp.zeros((BLK, 128 - 81), jnp.float32))
    feats = jnp.concatenate(cols, axis=1)
    x1 = lax.dot_general(feats, w1_ref[...], (((1,), (1,)), ((), ())),
                         preferred_element_type=jnp.float32) + b1_ref[...]
    x1_ref[...] = x1
    dist = dx * dx + dy * dy + dz * dz
    t97 = _kth_largest(dist, 97)
    mlt = dist < t97
    ones_col = jnp.ones((128, 1), jnp.float32)
    c_lt = lax.dot_general(jnp.where(mlt, 1.0, 0.0), ones_col,
                           (((1,), (0,)), ((), ())),
                           preferred_element_type=jnp.float32)
    meq = dist == t97
    corr_eq = jnp.where(meq, vals, -3.0e38)
    c_cut = _kth_largest(corr_eq, 32.0 - c_lt)
    mknn = mlt | (meq & (vals >= c_cut))
    mkf = jnp.where(mknn, 1.0, 0.0)
    f4 = (vals, dx, dy, dz)
    cnt = jnp.sum(mkf)
    sf = [jnp.sum(f * mkf) for f in f4]
    sff = [[jnp.sum(f4[i] * f4[j] * mkf) for j in range(4)] for i in range(4)]
    mx_ref[...] = jnp.zeros((BLK, 128), jnp.float32)
    mn_ref[...] = jnp.zeros((BLK, 128), jnp.float32)
    for c in range(64):
        u = (vals * wk_sm[c, 0] + dx * wk_sm[c, 1] + dy * wk_sm[c, 2]
             + dz * wk_sm[c, 3] + bk_sm[c])
        mx_ref[:, c:c + 1] = jnp.max(jnp.where(mknn, u, -3.0e38), axis=1,
                                     keepdims=True)
        mn_ref[:, c:c + 1] = jnp.min(jnp.where(mknn, u, 3.0e38), axis=1,
                                     keepdims=True)
    s1v = jnp.sum(x1, axis=0, keepdims=True)
    s2v = jnp.sum(x1 * x1, axis=0, keepdims=True)
    lane = lax.broadcasted_iota(jnp.int32, (1, 128), 1)
    knrow = jnp.zeros((1, 128), jnp.float32)
    for i in range(4):
        knrow = jnp.where(lane == i, sf[i], knrow)
    for i in range(4):
        for j in range(4):
            knrow = jnp.where(lane == 4 + 4 * i + j, sff[i][j], knrow)
    knrow = jnp.where(lane == 20, cnt, knrow)
    row = lax.broadcasted_iota(jnp.int32, (8, 128), 0)
    contrib = jnp.where(row == 0, s1v, jnp.where(row == 1, s2v,
                        jnp.where(row == 2, knrow, 0.0)))

    @pl.when(pl.program_id(0) == 0)
    def _():
        st_ref[...] = jnp.zeros((8, 128), jnp.float32)

    st_ref[...] += contrib


def _s3(vals, gx, gy, gz, cxb, cyb, czb, W1p, b1r, Wk, bk):
    grid = N // BLK
    bs = pl.BlockSpec((BLK, 128), lambda i: (i, 0))
    full = pl.BlockSpec((128, 128), lambda i: (0, 0))
    row1 = pl.BlockSpec((1, 128), lambda i: (0, 0))
    smem = pl.BlockSpec(memory_space=pltpu.SMEM)
    x1, mx, mn, st = pl.pallas_call(
        _s3_body,
        grid=(grid,),
        in_specs=[bs, bs, bs, bs, bs, bs, bs, full, row1, smem, smem],
        out_specs=[bs, bs, bs, pl.BlockSpec((8, 128), lambda i: (0, 0))],
        out_shape=[
            jax.ShapeDtypeStruct((N, 128), jnp.float32),
            jax.ShapeDtypeStruct((N, 128), jnp.float32),
            jax.ShapeDtypeStruct((N, 128), jnp.float32),
            jax.ShapeDtypeStruct((8, 128), jnp.float32),
        ],
    )(vals, gx, gy, gz, cxb, cyb, czb, W1p, b1r, Wk, bk)
    return x1, mx, mn, st


def _s4_body(x1_ref, mx_ref, mn_ref, st_ref, st_sm,
             g1_ref, be1_ref, p1_ref, w2_ref, b2_ref,
             wk_ref, bk_ref, gk_ref, bek_ref, pk_ref, wo_ref, bo_ref,
             out_ref):
    li = lax.broadcasted_iota(jnp.int32, (128, 128), 0)
    lj = lax.broadcasted_iota(jnp.int32, (128, 128), 1)
    lane = lax.broadcasted_iota(jnp.int32, (1, 128), 1)
    s1 = st_ref[0:1, :]
    s2 = st_ref[1:2, :]
    g16 = jnp.where((li // 16) == (lj // 16), 1.0, 0.0)
    gs1 = lax.dot_general(s1, g16, (((1,), (0,)), ((), ())),
                          preferred_element_type=jnp.float32)
    gs2 = lax.dot_general(s2, g16, (((1,), (0,)), ((), ())),
                          preferred_element_type=jnp.float32)
    denom = np.float32(16 * N)
    mean = gs1 / denom
    var = gs2 / denom - mean * mean
    a = g1_ref[...] * lax.rsqrt(var + 1e-5)
    d = be1_ref[...] - mean * a
    xh = x1_ref[...] * a + d
    xp = jnp.where(xh > 0, xh, p1_ref[...] * xh)
    vox = lax.dot_general(xp, w2_ref[...], (((1,), (1,)), ((), ())),
                          preferred_element_type=jnp.float32) + b2_ref[...]
    cnt = st_sm[2, 20]
    fsum = jnp.zeros((1, 128), jnp.float32)
    for i in range(4):
        fsum = jnp.where(lane == i, st_sm[2, i], fsum)
    sffm = jnp.zeros((128, 128), jnp.float32)
    for i in range(4):
        for j in range(4):
            sffm = jnp.where((li == i) & (lj == j), st_sm[2, 4 + 4 * i + j],
                             sffm)
    wk = wk_ref[...]
    bk = bk_ref[...]
    s1k_lin = lax.dot_general(fsum, wk, (((1,), (1,)), ((), ())),
                              preferred_element_type=jnp.float32)
    s1k = s1k_lin + cnt * bk
    t1 = lax.dot_general(wk, sffm, (((1,), (0,)), ((), ())),
                         preferred_element_type=jnp.float32)
    ones_row = jnp.ones((1, 128), jnp.float32)
    quad = lax.dot_general(ones_row, t1 * wk, (((1,), (1,)), ((), ())),
                           preferred_element_type=jnp.float32)
    s2k = quad + 2.0 * bk * s1k_lin + cnt * bk * bk
    g8 = jnp.where((li // 8) == (lj // 8), 1.0, 0.0)
    gk1 = lax.dot_general(s1k, g8, (((1,), (0,)), ((), ())),
                          preferred_element_type=jnp.float32)
    gk2 = lax.dot_general(s2k, g8, (((1,), (0,)), ((), ())),
                          preferred_element_type=jnp.float32)
    cdenom = 8.0 * cnt
    meank = gk1 / cdenom
    vark = gk2 / cdenom - meank * meank
    ak = gk_ref[...] * lax.rsqrt(vark + 1e-5)
    dk = bek_ref[...] - meank * ak
    z = jnp.where(ak > 0, mx_ref[...], mn_ref[...])
    zz = z * ak + dk
    zp = jnp.where(zz > 0, zz, pk_ref[...] * zz)
    knn = lax.dot_general(zp, wo_ref[...], (((1,), (1,)), ((), ())),
                          preferred_element_type=jnp.float32) + bo_ref[...]
    out_ref[...] = vox + knn


def _s4(x1, mx, mn, st, g1r, be1r, p1r, W2p, b2p, Wkp, bkp, gkp, bekp, pkp,
        Wop, bop):
    grid = N // BLK
    bs = pl.BlockSpec((BLK, 128), lambda i: (i, 0))
    full = pl.BlockSpec((128, 128), lambda i: (0, 0))
    row1 = pl.BlockSpec((1, 128), lambda i: (0, 0))
    st8 = pl.BlockSpec((8, 128), lambda i: (0, 0))
    smem = pl.BlockSpec(memory_space=pltpu.SMEM)
    out = pl.pallas_call(
        _s4_body,
        grid=(grid,),
        in_specs=[bs, bs, bs, st8, smem,
                  row1, row1, row1, full, row1,
                  full, row1, row1, row1, row1, full, row1],
        out_specs=bs,
        out_shape=jax.ShapeDtypeStruct((N, 128), jnp.float32),
    )(x1, mx, mn, st, st,
      g1r, be1r, p1r, W2p, b2p,
      Wkp, bkp, gkp, bekp, pkp, Wop, bop)
    return out


def _pad_rc(w, rows, cols):
    return jnp.zeros((rows, cols), w.dtype).at[:w.shape[0], :w.shape[1]].set(w)


def _pad_row(v, cols):
    return jnp.zeros((1, cols), v.dtype).at[0, :v.shape[0]].set(v)


def kernel(fmap1, fmap2, xyz2, coords, W1, b1, g1, be1, p1, W2, b2, Wk, bk, gk, bek, pk, Wo, bo):
    f1 = fmap1[0]
    f2 = fmap2[0]
    corr2d, th = _s1(f1, f2)
    xs = xyz2[0, :, 0]
    ys = xyz2[0, :, 1]
    zs = xyz2[0, :, 2]
    vals, gx, gy, gz = _s2(corr2d, th, xs, ys, zs)
    cxb = jnp.broadcast_to(coords[0, :, 0:1], (N, 128))
    cyb = jnp.broadcast_to(coords[0, :, 1:2], (N, 128))
    czb = jnp.broadcast_to(coords[0, :, 2:3], (N, 128))
    W1p = _pad_rc(W1, 128, 128)
    b1r = b1.reshape(1, 128)
    x1, mx, mn, st = _s3(vals, gx, gy, gz, cxb, cyb, czb, W1p, b1r, Wk, bk)
    out2d = _s4(
        x1, mx, mn, st,
        g1.reshape(1, 128), be1.reshape(1, 128),
        jnp.broadcast_to(p1.reshape(1, 1), (1, 128)),
        _pad_rc(W2, 128, 128), _pad_row(b2, 128),
        _pad_rc(Wk, 128, 128), _pad_row(bk, 128),
        _pad_row(gk, 128), _pad_row(bek, 128),
        jnp.broadcast_to(pk.reshape(1, 1), (1, 128)),
        _pad_rc(Wo, 128, 128), _pad_row(bo, 128),
    )
    return out2d[:, :64].T[None]
